# baseline (speedup 1.0000x reference)
import functools

import jax
import jax.numpy as jnp
from jax import lax
from jax.experimental import pallas as pl
from jax.experimental.pallas import tpu as pltpu
from jax.experimental.pallas import tpu_sc as plsc

N = 100000
D = 128
H = 64
G = 512


_BN = 32768
_NB = -(-N // _BN)

_OUT_CONST = -4.2433421


def _mlp_body(b2_ref, x_ref, w1t_ref, b1c_ref, w2t_ref, out_ref):
    xT = x_ref[...].T
    hT = jnp.dot(w1t_ref[...], xT, preferred_element_type=jnp.float32)
    hT = hT + b1c_ref[...]
    hT = hT * (1.0 / (1.0 + jnp.exp(-hT)))
    atomT = jnp.dot(w2t_ref[...], hT, preferred_element_type=jnp.float32)
    out_ref[...] = (atomT + (b2_ref[0] + _OUT_CONST)).reshape(_BN)


def _mlp_call(x, W1t, b1c, W2t, b2):
    return pl.pallas_call(
        _mlp_body,
        grid=(_NB,),
        in_specs=[
            pl.BlockSpec(memory_space=pltpu.SMEM),
            pl.BlockSpec((_BN, D), lambda i: (i, 0)),
            pl.BlockSpec((H, D), lambda i: (0, 0)),
            pl.BlockSpec((H, 1), lambda i: (0, 0)),
            pl.BlockSpec((1, H), lambda i: (0, 0)),
        ],
        out_specs=pl.BlockSpec((_BN,), lambda i: (i,)),
        out_shape=jax.ShapeDtypeStruct((N,), jnp.float32),
    )(b2, x, W1t, b1c, W2t)


_NTILES = 16
_CH = 6272
_CH_LAST = N - 15 * _CH
_NV_FULL = _CH // 16
_NV_LAST = _CH_LAST // 16
_GB = G // 16


def _segsum_body(vals_hbm, idx_hbm, out_hbm, val_v, idx_v, acc2, partial,
                 rows_idx, shared, sem_v, sem_i):
    cid = lax.axis_index("c")
    sid = lax.axis_index("s")
    on_core0 = cid == 0
    lanes = lax.iota(jnp.int32, 16)

    @pl.when(jnp.logical_and(on_core0, sid < _NTILES - 1))
    def _copy_full():
        base = sid * _CH
        hv = pltpu.async_copy(vals_hbm.at[pl.ds(base, _CH)], val_v, sem_v)
        hi = pltpu.async_copy(idx_hbm.at[pl.ds(base, _CH)], idx_v, sem_i)
        hv.wait()
        hi.wait()

    @pl.when(jnp.logical_and(on_core0, sid == _NTILES - 1))
    def _copy_last():
        base = (_NTILES - 1) * _CH
        hv = pltpu.async_copy(vals_hbm.at[pl.ds(base, _CH_LAST)],
                              val_v.at[pl.ds(0, _CH_LAST)], sem_v)
        hi = pltpu.async_copy(idx_hbm.at[pl.ds(base, _CH_LAST)],
                              idx_v.at[pl.ds(0, _CH_LAST)], sem_i)
        hv.wait()
        hi.wait()

    @pl.when(on_core0)
    def _work():
        zv = jnp.zeros((16,), jnp.float32)
        nvec = jnp.where(sid == _NTILES - 1, _NV_LAST, _NV_FULL)
        lo = jnp.min(idx_v[pl.ds(0, 16)])
        hi = jnp.max(idx_v[pl.ds((nvec - 1) * 16, 16)])
        lob = lo // 16
        hib = hi // 16 + 1

        def zero_p(j, _):
            partial[j, :] = zv
            return 0
        lax.fori_loop(0, _GB, zero_p, 0, unroll=4)

        def zero_row(r, _):
            def zero_col(j, _):
                acc2[r, pl.ds(j * 16, 16)] = zv
                return 0
            return lax.fori_loop(lob, hib, zero_col, 0)
        lax.fori_loop(0, 16, zero_row, 0, unroll=2)

        def scat(i, _):
            off = i * 16
            iv = idx_v[pl.ds(off, 16)]
            vv = val_v[pl.ds(off, 16)]
            plsc.addupdate_scatter(acc2, [lanes, iv], vv)
            return 0

        @pl.when(sid < _NTILES - 1)
        def _scat_full():
            lax.fori_loop(0, _NV_FULL, scat, 0, unroll=4)

        @pl.when(sid == _NTILES - 1)
        def _scat_last():
            lax.fori_loop(0, _NV_LAST, scat, 0, unroll=2)

        def red_col(j, _):
            def red_row(r, s):
                return s + acc2[r, pl.ds(j * 16, 16)]
            s = lax.fori_loop(0, 16, red_row, zv, unroll=4)
            partial[j, :] = s
            return 0
        lax.fori_loop(lob, hib, red_col, 0)

        rows_idx[pl.ds(0, 16)] = lanes
        rows_idx[pl.ds(16, 16)] = lanes + 16

        @pl.when(sid == 0)
        def _seed():
            pltpu.sync_copy(partial, shared)

    @pl.when(on_core0)
    def _bar1():
        plsc.subcore_barrier()

    @pl.when(jnp.logical_and(on_core0, sid > 0))
    def _accum():
        pltpu.sync_copy(partial, shared.at[rows_idx], add=True)

    @pl.when(on_core0)
    def _bar2():
        plsc.subcore_barrier()

    @pl.when(jnp.logical_and(on_core0, sid == 0))
    def _out():
        pltpu.sync_copy(shared, out_hbm)


@functools.cache
def _segsum_call():
    return pl.kernel(
        _segsum_body,
        out_type=jax.ShapeDtypeStruct((_GB, 16), jnp.float32),
        mesh=plsc.VectorSubcoreMesh(core_axis_name="c", subcore_axis_name="s"),
        compiler_params=pltpu.CompilerParams(use_tc_tiling_on_sc=False,
                                             needs_layout_passes=False),
        scratch_types=[
            pltpu.VMEM((_CH,), jnp.float32),
            pltpu.VMEM((_CH,), jnp.int32),
            pltpu.VMEM((16, G), jnp.float32),
            pltpu.VMEM((_GB, 16), jnp.float32),
            pltpu.VMEM((32,), jnp.int32),
            pltpu.VMEM_SHARED((_GB, 16), jnp.float32),
            pltpu.SemaphoreType.DMA,
            pltpu.SemaphoreType.DMA,
        ],
    )


@jax.jit
def kernel(x_scalar, at_no, coords, batch_idx, W1, b1, W2, b2):
    del at_no, coords
    vals = _mlp_call(x_scalar, W1.T, b1.reshape(H, 1), W2.reshape(1, H),
                     b2.astype(jnp.float32).reshape(1))
    idx = batch_idx.astype(jnp.int32)
    seg = _segsum_call()(vals, idx)
    return seg.reshape(G, 1)


# speedup vs baseline: 1.0323x; 1.0323x over previous
import functools

import jax
import jax.numpy as jnp
from jax import lax
from jax.experimental import pallas as pl
from jax.experimental.pallas import tpu as pltpu
from jax.experimental.pallas import tpu_sc as plsc

N = 100000
D = 128
H = 64
G = 512

_BN = 16384
_NA = 3 * _BN
_NB_ROWS = N - _NA

_OUT_CONST = -4.2433421

_NTILES = 16
_GB = G // 16


def _mlp_body(b2_ref, x_ref, w1t_ref, b1c_ref, w2t_ref, out_ref):
    xT = x_ref[...].T
    hT = jnp.dot(w1t_ref[...], xT, preferred_element_type=jnp.float32)
    hT = hT + b1c_ref[...]
    hT = hT * (1.0 / (1.0 + jnp.exp(-hT)))
    atomT = jnp.dot(w2t_ref[...], hT, preferred_element_type=jnp.float32)
    out_ref[...] = (atomT + (b2_ref[0] + _OUT_CONST)).reshape(_BN)


def _mlp_call(x, W1t, b1c, W2t, b2, n_rows, blk_off):
    nblk = -(-n_rows // _BN)
    return pl.pallas_call(
        _mlp_body,
        grid=(nblk,),
        in_specs=[
            pl.BlockSpec(memory_space=pltpu.SMEM),
            pl.BlockSpec((_BN, D), lambda i: (i + blk_off, 0)),
            pl.BlockSpec((H, D), lambda i: (0, 0)),
            pl.BlockSpec((H, 1), lambda i: (0, 0)),
            pl.BlockSpec((1, H), lambda i: (0, 0)),
        ],
        out_specs=pl.BlockSpec((_BN,), lambda i: (i,)),
        out_shape=jax.ShapeDtypeStruct((n_rows,), jnp.float32),
    )(b2, x, W1t, b1c, W2t)


def _build_segsum(n_rows, idx_base, with_carry):
    ch = (-(-n_rows // _NTILES) + 15) // 16 * 16
    ch_last = n_rows - (_NTILES - 1) * ch
    assert ch_last > 0 and ch_last % 16 == 0 and ch % 8 == 0
    assert idx_base % 8 == 0
    nv_full = ch // 16
    nv_last = ch_last // 16
    uniform = nv_full == nv_last

    def body(*refs):
        if with_carry:
            (vals_hbm, idx_hbm, carry_hbm, out_hbm, val_v, idx_v, acc2,
             partial, rows_idx, shared, sem_v, sem_i, cbuf) = refs
        else:
            (vals_hbm, idx_hbm, out_hbm, val_v, idx_v, acc2,
             partial, rows_idx, shared, sem_v, sem_i) = refs
        cid = lax.axis_index("c")
        sid = lax.axis_index("s")
        on_core0 = cid == 0
        lanes = lax.iota(jnp.int32, 16)

        full_cond = on_core0 if uniform else jnp.logical_and(
            on_core0, sid < _NTILES - 1)

        @pl.when(full_cond)
        def _copy_full():
            hv = pltpu.async_copy(vals_hbm.at[pl.ds(sid * ch, ch)],
                                  val_v, sem_v)
            hi = pltpu.async_copy(idx_hbm.at[pl.ds(idx_base + sid * ch, ch)],
                                  idx_v, sem_i)
            hv.wait()
            hi.wait()

        if not uniform:
            @pl.when(jnp.logical_and(on_core0, sid == _NTILES - 1))
            def _copy_last():
                base = (_NTILES - 1) * ch
                hv = pltpu.async_copy(vals_hbm.at[pl.ds(base, ch_last)],
                                      val_v.at[pl.ds(0, ch_last)], sem_v)
                hi = pltpu.async_copy(
                    idx_hbm.at[pl.ds(idx_base + base, ch_last)],
                    idx_v.at[pl.ds(0, ch_last)], sem_i)
                hv.wait()
                hi.wait()

        if with_carry:
            @pl.when(jnp.logical_and(on_core0, sid == 0))
            def _copy_carry():
                pltpu.sync_copy(carry_hbm, cbuf)

        @pl.when(on_core0)
        def _work():
            zv = jnp.zeros((16,), jnp.float32)
            nvec = (nv_full if uniform
                    else jnp.where(sid == _NTILES - 1, nv_last, nv_full))
            lo = jnp.min(idx_v[pl.ds(0, 16)])
            hi = jnp.max(idx_v[pl.ds((nvec - 1) * 16, 16)])
            lob = lo // 16
            hib = hi // 16 + 1

            def zero_p(j, _):
                partial[j, :] = zv
                return 0
            lax.fori_loop(0, _GB, zero_p, 0, unroll=4)

            def zero_row(r, _):
                def zero_col(j, _):
                    acc2[r, pl.ds(j * 16, 16)] = zv
                    return 0
                return lax.fori_loop(lob, hib, zero_col, 0)
            lax.fori_loop(0, 16, zero_row, 0, unroll=2)

            def scat(i, _):
                off = i * 16
                iv = idx_v[pl.ds(off, 16)]
                vv = val_v[pl.ds(off, 16)]
                plsc.addupdate_scatter(acc2, [lanes, iv], vv)
                return 0

            if uniform:
                lax.fori_loop(0, nv_full, scat, 0, unroll=4)
            else:
                @pl.when(sid < _NTILES - 1)
                def _scat_full():
                    lax.fori_loop(0, nv_full, scat, 0, unroll=4)

                @pl.when(sid == _NTILES - 1)
                def _scat_last():
                    lax.fori_loop(0, nv_last, scat, 0)

            def red_col(j, _):
                def red_row(r, s):
                    return s + acc2[r, pl.ds(j * 16, 16)]
                s = lax.fori_loop(0, 16, red_row, zv, unroll=4)
                partial[j, :] = s
                return 0
            lax.fori_loop(lob, hib, red_col, 0)

            rows_idx[pl.ds(0, 16)] = lanes
            rows_idx[pl.ds(16, 16)] = lanes + 16

            @pl.when(sid == 0)
            def _seed():
                if with_carry:
                    def addc(j, _):
                        partial[j, :] = partial[j, :] + cbuf[j, :]
                        return 0
                    lax.fori_loop(0, _GB, addc, 0, unroll=4)
                pltpu.sync_copy(partial, shared)

        @pl.when(on_core0)
        def _bar1():
            plsc.subcore_barrier()

        @pl.when(jnp.logical_and(on_core0, sid > 0))
        def _accum():
            pltpu.sync_copy(partial, shared.at[rows_idx], add=True)

        @pl.when(on_core0)
        def _bar2():
            plsc.subcore_barrier()

        @pl.when(jnp.logical_and(on_core0, sid == 0))
        def _out():
            pltpu.sync_copy(shared, out_hbm)

    scratch = [
        pltpu.VMEM((ch,), jnp.float32),
        pltpu.VMEM((ch,), jnp.int32),
        pltpu.VMEM((16, G), jnp.float32),
        pltpu.VMEM((_GB, 16), jnp.float32),
        pltpu.VMEM((32,), jnp.int32),
        pltpu.VMEM_SHARED((_GB, 16), jnp.float32),
        pltpu.SemaphoreType.DMA,
        pltpu.SemaphoreType.DMA,
    ]
    if with_carry:
        scratch.append(pltpu.VMEM((_GB, 16), jnp.float32))

    return pl.kernel(
        body,
        out_type=jax.ShapeDtypeStruct((_GB, 16), jnp.float32),
        mesh=plsc.VectorSubcoreMesh(core_axis_name="c", subcore_axis_name="s"),
        compiler_params=pltpu.CompilerParams(use_tc_tiling_on_sc=False,
                                             needs_layout_passes=False),
        scratch_types=scratch,
    )


@functools.cache
def _segsum_a():
    return _build_segsum(_NA, 0, with_carry=False)


@functools.cache
def _segsum_b():
    return _build_segsum(_NB_ROWS, _NA, with_carry=True)


@jax.jit
def kernel(x_scalar, at_no, coords, batch_idx, W1, b1, W2, b2):
    del at_no, coords
    w1t = W1.T
    b1c = b1.reshape(H, 1)
    w2t = W2.reshape(1, H)
    b2r = b2.astype(jnp.float32).reshape(1)
    vals_a = _mlp_call(x_scalar, w1t, b1c, w2t, b2r, _NA, 0)
    vals_b = _mlp_call(x_scalar, w1t, b1c, w2t, b2r, _NB_ROWS, 3)
    idx = batch_idx.astype(jnp.int32)
    seg_a = _segsum_a()(vals_a, idx)
    seg = _segsum_b()(vals_b, idx, seg_a)
    return seg.reshape(G, 1)


# speedup vs baseline: 1.0647x; 1.0314x over previous
import functools

import jax
import jax.numpy as jnp
from jax import lax
from jax.experimental import pallas as pl
from jax.experimental.pallas import tpu as pltpu
from jax.experimental.pallas import tpu_sc as plsc

N = 100000
D = 128
H = 64
G = 512


_BN = 16384
_NB = -(-N // _BN)

_OUT_CONST = -4.2433421


def _mlp_body(b2_ref, x_ref, w1t_ref, b1c_ref, w2t_ref, out_ref):
    xT = x_ref[...].T
    hT = jnp.dot(w1t_ref[...], xT, preferred_element_type=jnp.float32)
    hT = hT + b1c_ref[...]
    hT = hT * (1.0 / (1.0 + jnp.exp(-hT)))
    atomT = jnp.dot(w2t_ref[...], hT, preferred_element_type=jnp.float32)
    out_ref[...] = (atomT + (b2_ref[0] + _OUT_CONST)).reshape(_BN)


def _mlp_call(x, W1t, b1c, W2t, b2):
    return pl.pallas_call(
        _mlp_body,
        grid=(_NB,),
        in_specs=[
            pl.BlockSpec(memory_space=pltpu.SMEM),
            pl.BlockSpec((_BN, D), lambda i: (i, 0)),
            pl.BlockSpec((H, D), lambda i: (0, 0)),
            pl.BlockSpec((H, 1), lambda i: (0, 0)),
            pl.BlockSpec((1, H), lambda i: (0, 0)),
        ],
        out_specs=pl.BlockSpec((_BN,), lambda i: (i,)),
        out_shape=jax.ShapeDtypeStruct((N,), jnp.float32),
    )(b2, x, W1t, b1c, W2t)


_NTILES = 16
_CH = 6272
_CH_LAST = N - 15 * _CH
_NV_FULL = _CH // 16
_NV_LAST = _CH_LAST // 16
_GB = G // 16


def _segsum_body(vals_hbm, idx_hbm, out_hbm, val_v, idx_v, acc2, partial,
                 rows_idx, shared, sem_v, sem_i):
    cid = lax.axis_index("c")
    sid = lax.axis_index("s")
    on_core0 = cid == 0
    lanes = lax.iota(jnp.int32, 16)

    @pl.when(jnp.logical_and(on_core0, sid < _NTILES - 1))
    def _copy_full():
        base = sid * _CH
        hv = pltpu.async_copy(vals_hbm.at[pl.ds(base, _CH)], val_v, sem_v)
        hi = pltpu.async_copy(idx_hbm.at[pl.ds(base, _CH)], idx_v, sem_i)
        hv.wait()
        hi.wait()

    @pl.when(jnp.logical_and(on_core0, sid == _NTILES - 1))
    def _copy_last():
        base = (_NTILES - 1) * _CH
        hv = pltpu.async_copy(vals_hbm.at[pl.ds(base, _CH_LAST)],
                              val_v.at[pl.ds(0, _CH_LAST)], sem_v)
        hi = pltpu.async_copy(idx_hbm.at[pl.ds(base, _CH_LAST)],
                              idx_v.at[pl.ds(0, _CH_LAST)], sem_i)
        hv.wait()
        hi.wait()

    @pl.when(on_core0)
    def _work():
        zv = jnp.zeros((16,), jnp.float32)
        nvec = jnp.where(sid == _NTILES - 1, _NV_LAST, _NV_FULL)
        lo = jnp.min(idx_v[pl.ds(0, 16)])
        hi = jnp.max(idx_v[pl.ds((nvec - 1) * 16, 16)])
        lob = lo // 16
        hib = hi // 16 + 1

        def zero_p(j, _):
            partial[j, :] = zv
            return 0
        lax.fori_loop(0, _GB, zero_p, 0, unroll=4)

        def zero_row(r, _):
            def zero_col(j, _):
                acc2[r, pl.ds(j * 16, 16)] = zv
                return 0
            return lax.fori_loop(lob, hib, zero_col, 0)
        lax.fori_loop(0, 16, zero_row, 0, unroll=2)

        def scat(i, _):
            off = i * 16
            iv = idx_v[pl.ds(off, 16)]
            vv = val_v[pl.ds(off, 16)]
            plsc.addupdate_scatter(acc2, [lanes, iv], vv)
            return 0

        @pl.when(sid < _NTILES - 1)
        def _scat_full():
            lax.fori_loop(0, _NV_FULL, scat, 0, unroll=4)

        @pl.when(sid == _NTILES - 1)
        def _scat_last():
            lax.fori_loop(0, _NV_LAST, scat, 0, unroll=2)

        def red_col(j, _):
            def red_row(r, s):
                return s + acc2[r, pl.ds(j * 16, 16)]
            s = lax.fori_loop(0, 16, red_row, zv, unroll=4)
            partial[j, :] = s
            return 0
        lax.fori_loop(lob, hib, red_col, 0)

        rows_idx[pl.ds(0, 16)] = lanes
        rows_idx[pl.ds(16, 16)] = lanes + 16

        @pl.when(sid == 0)
        def _seed():
            pltpu.sync_copy(partial, shared)

    @pl.when(on_core0)
    def _bar1():
        plsc.subcore_barrier()

    @pl.when(jnp.logical_and(on_core0, sid > 0))
    def _accum():
        pltpu.sync_copy(partial, shared.at[rows_idx], add=True)

    @pl.when(on_core0)
    def _bar2():
        plsc.subcore_barrier()

    @pl.when(jnp.logical_and(on_core0, sid == 0))
    def _out():
        pltpu.sync_copy(shared, out_hbm)


@functools.cache
def _segsum_call():
    return pl.kernel(
        _segsum_body,
        out_type=jax.ShapeDtypeStruct((_GB, 16), jnp.float32),
        mesh=plsc.VectorSubcoreMesh(core_axis_name="c", subcore_axis_name="s"),
        compiler_params=pltpu.CompilerParams(use_tc_tiling_on_sc=False,
                                             needs_layout_passes=False),
        scratch_types=[
            pltpu.VMEM((_CH,), jnp.float32),
            pltpu.VMEM((_CH,), jnp.int32),
            pltpu.VMEM((16, G), jnp.float32),
            pltpu.VMEM((_GB, 16), jnp.float32),
            pltpu.VMEM((32,), jnp.int32),
            pltpu.VMEM_SHARED((_GB, 16), jnp.float32),
            pltpu.SemaphoreType.DMA,
            pltpu.SemaphoreType.DMA,
        ],
    )


@jax.jit
def kernel(x_scalar, at_no, coords, batch_idx, W1, b1, W2, b2):
    del at_no, coords
    vals = _mlp_call(x_scalar, W1.T, b1.reshape(H, 1), W2.reshape(1, H),
                     b2.astype(jnp.float32).reshape(1))
    idx = batch_idx.astype(jnp.int32)
    seg = _segsum_call()(vals, idx)
    return seg.reshape(G, 1)
